# baseline (speedup 1.0000x reference)
import functools

import jax
import jax.numpy as jnp
from jax import lax
from jax.experimental import pallas as pl
from jax.experimental.pallas import tpu as pltpu
from jax.experimental.pallas import tpu_sc as plsc

N_ATOMS = 10000
N_BONDS = 320000
N_TYPES = 20

_NC = 2
_NS = 16
_NW = _NC * _NS
_L = 16
_BLK = 128
_NBLK = N_BONDS // _BLK
_BPW = -(-_NBLK // _NW)
_E_PER = _BPW * _BLK


def _sqrt16(s):
    i = lax.bitcast_convert_type(s, jnp.int32)
    i = jnp.int32(0x5F3759DF) - lax.shift_right_logical(i, 1)
    r = lax.bitcast_convert_type(i, jnp.float32)
    half = s * jnp.float32(0.5)
    for _ in range(3):
        r = r * (jnp.float32(1.5) - half * r * r)
    return s * r


def _body(pos_h, typ_h, tk_h, map_h, out_h,
          pos_v, typ_v, tk_v, idx_v, out_v, sem):
    wid = lax.axis_index("s") * _NC + lax.axis_index("c")
    ebase = pl.multiple_of((wid * _NBLK) // _NW * _BLK, _BLK)

    copies = [
        pltpu.make_async_copy(pos_h, pos_v, sem),
        pltpu.make_async_copy(typ_h, typ_v, sem),
        pltpu.make_async_copy(tk_h, tk_v, sem),
        pltpu.make_async_copy(map_h.at[:, pl.ds(ebase, _E_PER)], idx_v, sem),
    ]
    for cp in copies:
        cp.start()
    for cp in copies:
        cp.wait()

    c0 = jnp.zeros((_L,), jnp.int32)
    c1 = jnp.full((_L,), 1, jnp.int32)
    c2 = jnp.full((_L,), 2, jnp.int32)

    def chunk(off):
        i = idx_v[0, pl.ds(off, _L)]
        j = idx_v[1, pl.ds(off, _L)]
        xi = plsc.load_gather(pos_v, [c0, i])
        yi = plsc.load_gather(pos_v, [c1, i])
        zi = plsc.load_gather(pos_v, [c2, i])
        xj = plsc.load_gather(pos_v, [c0, j])
        yj = plsc.load_gather(pos_v, [c1, j])
        zj = plsc.load_gather(pos_v, [c2, j])
        ti = plsc.load_gather(typ_v, [i])
        tj = plsc.load_gather(typ_v, [j])
        t = ti * N_TYPES + tj
        x0e = plsc.load_gather(tk_v, [t])
        ke = plsc.load_gather(tk_v, [t + jnp.int32(N_TYPES * N_TYPES)])
        dx = xi - xj
        dy = yi - yj
        dz = zi - zj
        s = dx * dx + dy * dy + dz * dz + jnp.float32(1e-12)
        d = _sqrt16(s)
        diff = d - x0e
        out_v[pl.ds(off, _L)] = ke * diff * diff

    half = (_E_PER // 2) // _BLK * _BLK
    plsc.parallel_loop(0, half, step=_L, unroll=8)(chunk)
    first = pltpu.make_async_copy(
        out_v.at[pl.ds(0, half)], out_h.at[pl.ds(ebase, half)], sem)
    first.start()
    plsc.parallel_loop(half, _E_PER, step=_L, unroll=8)(chunk)
    first.wait()
    pltpu.sync_copy(out_v.at[pl.ds(half, _E_PER - half)],
                    out_h.at[pl.ds(ebase + half, _E_PER - half)])


@functools.partial(
    pl.kernel,
    mesh=plsc.VectorSubcoreMesh(core_axis_name="c", subcore_axis_name="s"),
    out_type=jax.ShapeDtypeStruct((N_BONDS,), jnp.float32),
    compiler_params=pltpu.CompilerParams(needs_layout_passes=False),
    scratch_types=[
        pltpu.VMEM((3, N_ATOMS), jnp.float32),
        pltpu.VMEM((N_ATOMS,), jnp.int32),
        pltpu.VMEM((2 * N_TYPES * N_TYPES,), jnp.float32),
        pltpu.VMEM((2, _E_PER), jnp.int32),
        pltpu.VMEM((_E_PER,), jnp.float32),
        pltpu.SemaphoreType.DMA,
    ],
)
def _harmonic_sc(pos_t, typ, tk, mapping, out,
                 pos_v, typ_v, tk_v, idx_v, out_v, sem):
    _body(pos_t, typ, tk, mapping, out,
          pos_v, typ_v, tk_v, idx_v, out_v, sem)


def kernel(pos, mapping, atom_types, x_0, k_const):
    pos_t = pos.astype(jnp.float32).T
    typ = atom_types.astype(jnp.int32)
    mp = mapping.astype(jnp.int32)
    tk = jnp.concatenate([x_0.astype(jnp.float32).reshape(-1),
                          k_const.astype(jnp.float32).reshape(-1)])
    return _harmonic_sc(pos_t, typ, tk, mp)


# speedup vs baseline: 1.0178x; 1.0178x over previous
import functools

import jax
import jax.numpy as jnp
from jax import lax
from jax.experimental import pallas as pl
from jax.experimental.pallas import tpu as pltpu
from jax.experimental.pallas import tpu_sc as plsc

N_ATOMS = 10000
N_BONDS = 320000
N_TYPES = 20

_NC = 2
_NS = 16
_NW = _NC * _NS
_L = 16
_BLK = 128
_NBLK = N_BONDS // _BLK
_BPW = -(-_NBLK // _NW)
_E_PER = _BPW * _BLK


def _sqrt16(s):
    i = lax.bitcast_convert_type(s, jnp.int32)
    i = jnp.int32(0x5F3759DF) - lax.shift_right_logical(i, 1)
    r = lax.bitcast_convert_type(i, jnp.float32)
    half = s * jnp.float32(0.5)
    for _ in range(3):
        r = r * (jnp.float32(1.5) - half * r * r)
    return s * r


def _body(pos_h, typ_h, tk_h, map_h, out_h,
          pos_v, typ_v, tk_v, idx_v, out_v, sem):
    wid = lax.axis_index("s") * _NC + lax.axis_index("c")
    ebase = pl.multiple_of((wid * _NBLK) // _NW * _BLK, _BLK)

    copies = [
        pltpu.make_async_copy(pos_h, pos_v, sem),
        pltpu.make_async_copy(typ_h, typ_v, sem),
        pltpu.make_async_copy(tk_h, tk_v, sem),
        pltpu.make_async_copy(map_h.at[:, pl.ds(ebase, _E_PER)], idx_v, sem),
    ]
    for cp in copies:
        cp.start()
    for cp in copies:
        cp.wait()

    c0 = jnp.zeros((_L,), jnp.int32)
    c1 = jnp.full((_L,), 1, jnp.int32)
    c2 = jnp.full((_L,), 2, jnp.int32)

    def chunk(off):
        i = idx_v[0, pl.ds(off, _L)]
        j = idx_v[1, pl.ds(off, _L)]
        xi = plsc.load_gather(pos_v, [c0, i])
        yi = plsc.load_gather(pos_v, [c1, i])
        zi = plsc.load_gather(pos_v, [c2, i])
        xj = plsc.load_gather(pos_v, [c0, j])
        yj = plsc.load_gather(pos_v, [c1, j])
        zj = plsc.load_gather(pos_v, [c2, j])
        ti = plsc.load_gather(typ_v, [i])
        tj = plsc.load_gather(typ_v, [j])
        t = ti * N_TYPES + tj
        x0e = plsc.load_gather(tk_v, [t])
        ke = plsc.load_gather(tk_v, [t + jnp.int32(N_TYPES * N_TYPES)])
        dx = xi - xj
        dy = yi - yj
        dz = zi - zj
        s = dx * dx + dy * dy + dz * dz + jnp.float32(1e-12)
        d = _sqrt16(s)
        diff = d - x0e
        out_v[pl.ds(off, _L)] = ke * diff * diff

    plsc.parallel_loop(0, _E_PER, step=_L, unroll=8)(chunk)
    pltpu.sync_copy(out_v, out_h.at[pl.ds(ebase, _E_PER)])


@functools.partial(
    pl.kernel,
    mesh=plsc.VectorSubcoreMesh(core_axis_name="c", subcore_axis_name="s"),
    out_type=jax.ShapeDtypeStruct((N_BONDS,), jnp.float32),
    compiler_params=pltpu.CompilerParams(needs_layout_passes=False),
    scratch_types=[
        pltpu.VMEM((3, N_ATOMS), jnp.float32),
        pltpu.VMEM((N_ATOMS,), jnp.int32),
        pltpu.VMEM((2 * N_TYPES * N_TYPES,), jnp.float32),
        pltpu.VMEM((2, _E_PER), jnp.int32),
        pltpu.VMEM((_E_PER,), jnp.float32),
        pltpu.SemaphoreType.DMA,
    ],
)
def _harmonic_sc(pos_t, typ, tk, mapping, out,
                 pos_v, typ_v, tk_v, idx_v, out_v, sem):
    _body(pos_t, typ, tk, mapping, out,
          pos_v, typ_v, tk_v, idx_v, out_v, sem)


def kernel(pos, mapping, atom_types, x_0, k_const):
    pos_t = pos.astype(jnp.float32).T
    typ = atom_types.astype(jnp.int32)
    mp = mapping.astype(jnp.int32)
    tk = jnp.concatenate([x_0.astype(jnp.float32).reshape(-1),
                          k_const.astype(jnp.float32).reshape(-1)])
    return _harmonic_sc(pos_t, typ, tk, mp)
